# Initial kernel scaffold; baseline (speedup 1.0000x reference)
import jax
import jax.numpy as jnp
from jax.experimental import pallas as pl


def kernel(discrete, continuous, logits, locs, scales):
    raise NotImplementedError("write your pallas kernel here")


# speedup vs baseline: 1.4531x; 1.4531x over previous
import functools
import math

import jax
import jax.numpy as jnp
from jax import lax
from jax.experimental import pallas as pl
from jax.experimental.pallas import tpu as pltpu
from jax.experimental.pallas import tpu_sc as plsc

_SUPPORT = 1_000_000
_BATCH = 16_384
_NC = 2
_NS = 16
_NW = _NC * _NS
_BPW = _BATCH // _NW
_CHUNK = 128
_NCHUNK = _BPW // _CHUNK

_LANES = 128
_PAD_ROWS = 7816
_PAD = _PAD_ROWS * _LANES

_HALF_LOG_2PI = 0.5 * math.log(2.0 * math.pi)


def _sc_gather(disc, logits, locs, scales):
    mesh = plsc.VectorSubcoreMesh(core_axis_name="c", subcore_axis_name="s")

    @functools.partial(
        pl.kernel,
        mesh=mesh,
        out_type=(jax.ShapeDtypeStruct((_BATCH,), jnp.float32),) * 3,
        scratch_types=[
            pltpu.VMEM((_NCHUNK, _CHUNK), jnp.int32),
            pltpu.VMEM((_NCHUNK, _CHUNK), jnp.float32),
            pltpu.VMEM((_NCHUNK, _CHUNK), jnp.float32),
            pltpu.VMEM((_NCHUNK, _CHUNK), jnp.float32),
            pltpu.SemaphoreType.DMA,
        ],
    )
    def k(disc_h, logits_h, locs_h, scales_h, glog_h, gloc_h, gscl_h,
          idx_v, a_v, b_v, c_v, sem):
        wid = lax.axis_index("s") * _NC + lax.axis_index("c")
        base = wid * _BPW
        for j in range(_NCHUNK):
            pltpu.sync_copy(disc_h.at[pl.ds(base + j * _CHUNK, _CHUNK)],
                            idx_v.at[j])
        handles = []
        for j in range(_NCHUNK):
            handles.append(pltpu.async_copy(logits_h.at[idx_v.at[j]],
                                            a_v.at[j], sem))
            handles.append(pltpu.async_copy(locs_h.at[idx_v.at[j]],
                                            b_v.at[j], sem))
            handles.append(pltpu.async_copy(scales_h.at[idx_v.at[j]],
                                            c_v.at[j], sem))
        for h in handles:
            h.wait()
        for j in range(_NCHUNK):
            off = pl.ds(base + j * _CHUNK, _CHUNK)
            pltpu.sync_copy(a_v.at[j], glog_h.at[off])
            pltpu.sync_copy(b_v.at[j], gloc_h.at[off])
            pltpu.sync_copy(c_v.at[j], gscl_h.at[off])

    return k(disc, logits, locs, scales)


def _lse_body(x_ref, o_ref):
    v = x_ref[...]
    m = jnp.max(v)
    o_ref[0] = m + jnp.log(jnp.sum(jnp.exp(v - m)))


def _lse(logits):
    x = jnp.concatenate(
        [logits, jnp.full((_PAD - _SUPPORT,), -1e30, jnp.float32)]
    ).reshape(_PAD_ROWS, _LANES)
    return pl.pallas_call(
        _lse_body,
        out_shape=jax.ShapeDtypeStruct((1,), jnp.float32),
        in_specs=[pl.BlockSpec(memory_space=pltpu.VMEM)],
        out_specs=pl.BlockSpec(memory_space=pltpu.SMEM),
    )(x)


def _combine_body(logz_ref, glog_ref, gloc_ref, gscl_ref, cont_ref, o_ref):
    z = (cont_ref[...] - gloc_ref[...]) / gscl_ref[...]
    o_ref[...] = (glog_ref[...] - logz_ref[0] - 0.5 * z * z
                  - jnp.log(gscl_ref[...]) - _HALF_LOG_2PI)


def _combine(logz, glog, gloc, gscl, cont):
    return pl.pallas_call(
        _combine_body,
        out_shape=jax.ShapeDtypeStruct((_BATCH,), jnp.float32),
        in_specs=[pl.BlockSpec(memory_space=pltpu.SMEM)]
                 + [pl.BlockSpec(memory_space=pltpu.VMEM)] * 4,
        out_specs=pl.BlockSpec(memory_space=pltpu.VMEM),
    )(logz, glog, gloc, gscl, cont)


def kernel(discrete, continuous, logits, locs, scales):
    disc = discrete.astype(jnp.int32)
    glog, gloc, gscl = _sc_gather(disc, logits, locs, scales)
    logz = _lse(logits)
    return _combine(logz, glog, gloc, gscl, continuous)


# speedup vs baseline: 1.5221x; 1.0475x over previous
import functools
import math

import jax
import jax.numpy as jnp
from jax import lax
from jax.experimental import pallas as pl
from jax.experimental.pallas import tpu as pltpu
from jax.experimental.pallas import tpu_sc as plsc

_SUPPORT = 1_000_000
_BATCH = 16_384
_NC = 2
_NS = 16
_NW = _NC * _NS
_BPW = _BATCH // _NW
_CHUNK = 128
_NCHUNK = _BPW // _CHUNK

_LANES = 128
_PAD_ROWS = 7816
_PAD = _PAD_ROWS * _LANES

_HALF_LOG_2PI = 0.5 * math.log(2.0 * math.pi)


def _sc_gather(disc, logits, locs, scales):
    mesh = plsc.VectorSubcoreMesh(core_axis_name="c", subcore_axis_name="s")

    @functools.partial(
        pl.kernel,
        mesh=mesh,
        out_type=(jax.ShapeDtypeStruct((_BATCH,), jnp.float32),) * 3,
        scratch_types=[
            pltpu.VMEM((_BPW,), jnp.int32),
            pltpu.VMEM((_BPW,), jnp.float32),
            pltpu.VMEM((_BPW,), jnp.float32),
            pltpu.VMEM((_BPW,), jnp.float32),
            pltpu.SemaphoreType.DMA,
            pltpu.SemaphoreType.DMA,
        ],
    )
    def k(disc_h, logits_h, locs_h, scales_h, glog_h, gloc_h, gscl_h,
          idx_v, a_v, b_v, c_v, gsem, osem):
        wid = lax.axis_index("s") * _NC + lax.axis_index("c")
        base = wid * _BPW
        pltpu.sync_copy(disc_h.at[pl.ds(base, _BPW)], idx_v)
        handles = []
        for j in range(_NCHUNK):
            sl = pl.ds(j * _CHUNK, _CHUNK)
            handles.append(pltpu.async_copy(logits_h.at[idx_v.at[sl]],
                                            a_v.at[sl], gsem))
            handles.append(pltpu.async_copy(locs_h.at[idx_v.at[sl]],
                                            b_v.at[sl], gsem))
            handles.append(pltpu.async_copy(scales_h.at[idx_v.at[sl]],
                                            c_v.at[sl], gsem))
        for h in handles:
            h.wait()
        out = pl.ds(base, _BPW)
        oh = [pltpu.async_copy(a_v, glog_h.at[out], osem),
              pltpu.async_copy(b_v, gloc_h.at[out], osem),
              pltpu.async_copy(c_v, gscl_h.at[out], osem)]
        for h in oh:
            h.wait()

    return k(disc, logits, locs, scales)


def _lse_body(x_ref, o_ref):
    v = x_ref[...]
    m = jnp.max(v)
    o_ref[0] = m + jnp.log(jnp.sum(jnp.exp(v - m)))


def _lse(logits):
    x = jnp.concatenate(
        [logits, jnp.full((_PAD - _SUPPORT,), -1e30, jnp.float32)]
    ).reshape(_PAD_ROWS, _LANES)
    return pl.pallas_call(
        _lse_body,
        out_shape=jax.ShapeDtypeStruct((1,), jnp.float32),
        in_specs=[pl.BlockSpec(memory_space=pltpu.VMEM)],
        out_specs=pl.BlockSpec(memory_space=pltpu.SMEM),
    )(x)


def _combine_body(logz_ref, glog_ref, gloc_ref, gscl_ref, cont_ref, o_ref):
    z = (cont_ref[...] - gloc_ref[...]) / gscl_ref[...]
    o_ref[...] = (glog_ref[...] - logz_ref[0] - 0.5 * z * z
                  - jnp.log(gscl_ref[...]) - _HALF_LOG_2PI)


def _combine(logz, glog, gloc, gscl, cont):
    return pl.pallas_call(
        _combine_body,
        out_shape=jax.ShapeDtypeStruct((_BATCH,), jnp.float32),
        in_specs=[pl.BlockSpec(memory_space=pltpu.SMEM)]
                 + [pl.BlockSpec(memory_space=pltpu.VMEM)] * 4,
        out_specs=pl.BlockSpec(memory_space=pltpu.VMEM),
    )(logz, glog, gloc, gscl, cont)


def kernel(discrete, continuous, logits, locs, scales):
    disc = discrete.astype(jnp.int32)
    glog, gloc, gscl = _sc_gather(disc, logits, locs, scales)
    logz = _lse(logits)
    return _combine(logz, glog, gloc, gscl, continuous)
